# baseline (speedup 1.0000x reference)
import jax
import jax.numpy as jnp
from jax.experimental import pallas as pl
from jax.experimental.pallas import tpu as pltpu
from jax.experimental.pallas import tpu_sc as plsc

N = 10000
H = 32
D_IN = 128
D_OUT = 128
K = 15
KP_EXTENT = 0.12

NP = 10240
EP = NP * H
PACK = 256
GW = 128
SC_STEPS = EP // GW
B = 128
GRID = NP // B


def _sc_gather(packed, idx_flat):
    vector_mesh = plsc.VectorSubcoreMesh(
        core_axis_name="c", subcore_axis_name="s")

    @pl.kernel(
        out_type=jax.ShapeDtypeStruct((EP, PACK), packed.dtype),
        mesh=vector_mesh,
    )
    def gather_kernel(x_hbm, i_hbm, o_hbm):
        def body(i_vmem, o_vmem):
            pltpu.sync_copy(x_hbm.at[i_vmem.at[0]], o_vmem)

        pltpu.emit_pipeline(
            body,
            grid=(SC_STEPS,),
            in_specs=[pl.BlockSpec((1, GW), lambda i: (0, i))],
            out_specs=[pl.BlockSpec((GW, PACK), lambda i: (i, 0))],
            core_axis_name=("c", "s"),
            dimension_semantics=(pltpu.PARALLEL,),
        )(i_hbm, o_hbm)

    return gather_kernel(packed, idx_flat)


def _tc_body(gx_ref, pts_ref, kp_ref, w2_ref, out_ref):
    g3 = gx_ref[...].reshape(B, H, PACK)
    xg = g3[:, :, :D_IN]
    pn = g3[:, :, D_IN:D_IN + 3]
    rel = pn - pts_ref[...][:, None, :]
    inv_ext = 1.0 / KP_EXTENT
    wfs = []
    for k in range(K):
        dx = rel[:, :, 0] - kp_ref[k, 0]
        dy = rel[:, :, 1] - kp_ref[k, 1]
        dz = rel[:, :, 2] - kp_ref[k, 2]
        d2 = dx * dx + dy * dy + dz * dz
        wk = jnp.maximum(1.0 - jnp.sqrt(d2) * inv_ext, 0.0)
        wfs.append(jnp.sum(wk[:, :, None] * xg, axis=1))
    wf = jnp.concatenate(wfs, axis=1)
    out = jnp.dot(wf, w2_ref[...], preferred_element_type=jnp.float32)
    out_ref[...] = jnp.where(out > 0, out, 0.1 * out)


def _tc_compute(gx, pts_pad, kernel_points, w2):
    return pl.pallas_call(
        _tc_body,
        grid=(GRID,),
        in_specs=[
            pl.BlockSpec((B * H, PACK), lambda i: (i, 0)),
            pl.BlockSpec((B, 3), lambda i: (i, 0)),
            pl.BlockSpec(memory_space=pltpu.SMEM),
            pl.BlockSpec((K * D_IN, D_OUT), lambda i: (0, 0)),
        ],
        out_specs=pl.BlockSpec((B, D_OUT), lambda i: (i, 0)),
        out_shape=jax.ShapeDtypeStruct((NP, D_OUT), jnp.float32),
    )(gx, pts_pad, kernel_points, w2)


def kernel(points, features, neighbor_indices, kernel_points, weights):
    packed = jnp.concatenate(
        [features, points, jnp.zeros((N, PACK - D_IN - 3), jnp.float32)],
        axis=1)
    idx_flat = jnp.pad(
        neighbor_indices, ((0, NP - N), (0, 0))).reshape(1, EP)
    pts_pad = jnp.pad(points, ((0, NP - N), (0, 0)))
    w2 = weights.reshape(K * D_IN, D_OUT)
    gx = _sc_gather(packed, idx_flat)
    out = _tc_compute(gx, pts_pad, kernel_points, w2)
    return out[:N]


# speedup vs baseline: 2.3406x; 2.3406x over previous
import functools

import jax
import jax.numpy as jnp
from jax import lax
from jax.experimental import pallas as pl
from jax.experimental.pallas import tpu as pltpu
from jax.experimental.pallas import tpu_sc as plsc

N = 10000
H = 32
D_IN = 128
D_OUT = 128
K = 15
KP_EXTENT = 0.12

NP = 10240
EP = NP * H
NW = 32
EW = EP // NW
CH = 2048
GW = 256
SC_STEPS = EP // GW
B = 256
GRID = NP // B

_MESH = dict(core_axis_name="c", subcore_axis_name="s")


def _sc_weights(px, py, pz, idx_flat, kp_rep):
    mesh = plsc.VectorSubcoreMesh(**_MESH)

    @functools.partial(
        pl.kernel,
        out_type=jax.ShapeDtypeStruct((EP * 16,), jnp.float32),
        mesh=mesh,
        compiler_params=pltpu.CompilerParams(needs_layout_passes=False),
        scratch_types=[
            pltpu.VMEM((N,), jnp.float32),
            pltpu.VMEM((N,), jnp.float32),
            pltpu.VMEM((N,), jnp.float32),
            pltpu.VMEM((EW,), jnp.int32),
            pltpu.VMEM((720,), jnp.float32),
            pltpu.VMEM((CH * 16,), jnp.float32),
        ],
    )
    def wkern(px_h, py_h, pz_h, idx_h, kp_h, w_h,
              px_v, py_v, pz_v, idx_v, kp_v, out_v):
        wid = lax.axis_index("s") * 2 + lax.axis_index("c")
        base = wid * EW
        pltpu.sync_copy(px_h, px_v)
        pltpu.sync_copy(py_h, py_v)
        pltpu.sync_copy(pz_h, pz_v)
        pltpu.sync_copy(idx_h.at[pl.ds(base, EW)], idx_v)
        pltpu.sync_copy(kp_h, kp_v)

        kvecs = [
            (kp_v[pl.ds(k * 48, 16)],
             kp_v[pl.ds(k * 48 + 16, 16)],
             kp_v[pl.ds(k * 48 + 32, 16)])
            for k in range(K)
        ]
        lane = lax.iota(jnp.int32, 16)
        inv_ext = jnp.float32(1.0 / KP_EXTENT)

        @pl.loop(0, EW, step=CH)
        def _chunk(c0):
            @pl.loop(0, CH, step=16)
            def _vec(v0):
                i_nbr = idx_v[pl.ds(c0 + v0, 16)]
                e_g = base + c0 + v0 + lane
                i_ctr = lax.shift_right_logical(e_g, 5)
                xn = plsc.load_gather(px_v, [i_nbr])
                yn = plsc.load_gather(py_v, [i_nbr])
                zn = plsc.load_gather(pz_v, [i_nbr])
                xc = plsc.load_gather(px_v, [i_ctr])
                yc = plsc.load_gather(py_v, [i_ctr])
                zc = plsc.load_gather(pz_v, [i_ctr])
                rx = xn - xc
                ry = yn - yc
                rz = zn - zc
                row16 = (v0 + lane) * 16
                for k in range(K):
                    kx, ky, kz = kvecs[k]
                    dx = rx - kx
                    dy = ry - ky
                    dz = rz - kz
                    d2 = jnp.maximum(dx * dx + dy * dy + dz * dz,
                                     jnp.float32(1e-24))
                    bits = plsc.bitcast(d2, jnp.int32)
                    seed = jnp.int32(0x5F3759DF) - lax.shift_right_logical(
                        bits, 1)
                    r = plsc.bitcast(seed, jnp.float32)
                    for _ in range(3):
                        r = r * (jnp.float32(1.5)
                                 - jnp.float32(0.5) * d2 * r * r)
                    dist = d2 * r
                    w = jnp.maximum(jnp.float32(1.0) - dist * inv_ext,
                                    jnp.float32(0.0))
                    plsc.store_scatter(out_v, [row16 + k], w)

            pltpu.sync_copy(out_v, w_h.at[pl.ds((base + c0) * 16, CH * 16)])

    return wkern(px, py, pz, idx_flat, kp_rep)


def _sc_gather(features, idx_2d):
    mesh = plsc.VectorSubcoreMesh(**_MESH)

    @functools.partial(
        pl.kernel,
        out_type=jax.ShapeDtypeStruct((EP, D_IN), features.dtype),
        mesh=mesh,
    )
    def gather_kernel(x_hbm, i_hbm, o_hbm):
        def body(i_vmem, o_vmem):
            pltpu.sync_copy(x_hbm.at[i_vmem.at[0]], o_vmem)

        pltpu.emit_pipeline(
            body,
            grid=(SC_STEPS,),
            in_specs=[pl.BlockSpec((1, GW), lambda i: (0, i))],
            out_specs=[pl.BlockSpec((GW, D_IN), lambda i: (i, 0))],
            core_axis_name=("c", "s"),
            dimension_semantics=(pltpu.PARALLEL,),
        )(i_hbm, o_hbm)

    return gather_kernel(features, idx_2d)


def _tc_body(gx_ref, wt_ref, w2_ref, out_ref):
    feats = gx_ref[...]
    wfs = []
    for k in range(K):
        wcol = wt_ref[:, k:k + 1]
        scaled = feats * wcol
        wfs.append(scaled.reshape(B, H, D_IN).sum(axis=1))
    wf = jnp.concatenate(wfs, axis=1)
    out = jnp.dot(wf, w2_ref[...], preferred_element_type=jnp.float32)
    out_ref[...] = jnp.where(out > 0, out, 0.1 * out)


def _tc_compute(gx, wt, w2):
    return pl.pallas_call(
        _tc_body,
        grid=(GRID,),
        in_specs=[
            pl.BlockSpec((B * H, D_IN), lambda i: (i, 0)),
            pl.BlockSpec((B * H, 16), lambda i: (i, 0)),
            pl.BlockSpec((K * D_IN, D_OUT), lambda i: (0, 0)),
        ],
        out_specs=pl.BlockSpec((B, D_OUT), lambda i: (i, 0)),
        out_shape=jax.ShapeDtypeStruct((NP, D_OUT), jnp.float32),
    )(gx, wt, w2)


def kernel(points, features, neighbor_indices, kernel_points, weights):
    px = points[:, 0]
    py = points[:, 1]
    pz = points[:, 2]
    idx_pad = jnp.pad(neighbor_indices, ((0, NP - N), (0, 0)))
    idx_flat = idx_pad.reshape(EP)
    kp_rep = jnp.tile(kernel_points.reshape(K * 3, 1), (1, 16)).reshape(720)
    w2 = weights.reshape(K * D_IN, D_OUT)
    wt = _sc_weights(px, py, pz, idx_flat, kp_rep).reshape(EP, 16)
    gx = _sc_gather(features, idx_flat.reshape(1, EP))
    out = _tc_compute(gx, wt, w2)
    return out[:N]


# speedup vs baseline: 2.3533x; 1.0054x over previous
import functools

import jax
import jax.numpy as jnp
from jax import lax
from jax.experimental import pallas as pl
from jax.experimental.pallas import tpu as pltpu
from jax.experimental.pallas import tpu_sc as plsc

N = 10000
H = 32
D_IN = 128
D_OUT = 128
K = 15
KP_EXTENT = 0.12

NP = 10240
EP = NP * H
NW = 32
EW = EP // NW
CHU = 256
NCH = EW // CHU
B = 256
GRID = NP // B

_MESH = dict(core_axis_name="c", subcore_axis_name="s")


def _sc_fused(feats, px, py, pz, idx_flat, kp_rep):
    mesh = plsc.VectorSubcoreMesh(**_MESH)

    @functools.partial(
        pl.kernel,
        out_type=(
            jax.ShapeDtypeStruct((EP, D_IN), jnp.float32),
            jax.ShapeDtypeStruct((EP * 16,), jnp.float32),
        ),
        mesh=mesh,
        compiler_params=pltpu.CompilerParams(needs_layout_passes=False),
        scratch_types=[
            pltpu.VMEM((N,), jnp.float32),
            pltpu.VMEM((N,), jnp.float32),
            pltpu.VMEM((N,), jnp.float32),
            pltpu.VMEM((EW,), jnp.int32),
            pltpu.VMEM((720,), jnp.float32),
            pltpu.VMEM((CHU * 16,), jnp.float32),
            pltpu.VMEM((CHU, D_IN), jnp.float32),
            pltpu.VMEM((CHU, D_IN), jnp.float32),
            pltpu.VMEM((CHU,), jnp.int32),
            pltpu.VMEM((CHU,), jnp.int32),
            pltpu.SemaphoreType.DMA,
            pltpu.SemaphoreType.DMA,
            pltpu.SemaphoreType.DMA,
            pltpu.SemaphoreType.DMA,
        ],
    )
    def fused(feat_h, px_h, py_h, pz_h, idx_h, kp_h, gx_h, w_h,
              px_v, py_v, pz_v, idx_v, kp_v, wout_v,
              rows0, rows1, idxc0, idxc1,
              sin0, sin1, sout0, sout1):
        wid = lax.axis_index("s") * 2 + lax.axis_index("c")
        base = wid * EW
        pltpu.sync_copy(px_h, px_v)
        pltpu.sync_copy(py_h, py_v)
        pltpu.sync_copy(pz_h, pz_v)
        pltpu.sync_copy(idx_h.at[pl.ds(base, EW)], idx_v)
        pltpu.sync_copy(kp_h, kp_v)

        kvecs = [
            (kp_v[pl.ds(k * 48, 16)],
             kp_v[pl.ds(k * 48 + 16, 16)],
             kp_v[pl.ds(k * 48 + 32, 16)])
            for k in range(K)
        ]
        lane = lax.iota(jnp.int32, 16)
        inv_ext = jnp.float32(1.0 / KP_EXTENT)
        rows = (rows0, rows1)
        idxc = (idxc0, idxc1)
        sin = (sin0, sin1)
        sout = (sout0, sout1)

        def weights_for(c0):
            @pl.loop(0, CHU, step=16)
            def _vec(v0):
                i_nbr = idx_v[pl.ds(c0 + v0, 16)]
                e_g = base + c0 + v0 + lane
                i_ctr = lax.shift_right_logical(e_g, 5)
                xn = plsc.load_gather(px_v, [i_nbr])
                yn = plsc.load_gather(py_v, [i_nbr])
                zn = plsc.load_gather(pz_v, [i_nbr])
                xc = plsc.load_gather(px_v, [i_ctr])
                yc = plsc.load_gather(py_v, [i_ctr])
                zc = plsc.load_gather(pz_v, [i_ctr])
                rx = xn - xc
                ry = yn - yc
                rz = zn - zc
                row16 = (v0 + lane) * 16
                for k in range(K):
                    kx, ky, kz = kvecs[k]
                    dx = rx - kx
                    dy = ry - ky
                    dz = rz - kz
                    d2 = jnp.maximum(dx * dx + dy * dy + dz * dz,
                                     jnp.float32(1e-24))
                    bits = plsc.bitcast(d2, jnp.int32)
                    seed = jnp.int32(0x5F3759DF) - lax.shift_right_logical(
                        bits, 1)
                    r = plsc.bitcast(seed, jnp.float32)
                    for _ in range(3):
                        r = r * (jnp.float32(1.5)
                                 - jnp.float32(0.5) * d2 * r * r)
                    dist = d2 * r
                    w = jnp.maximum(jnp.float32(1.0) - dist * inv_ext,
                                    jnp.float32(0.0))
                    plsc.store_scatter(wout_v, [row16 + k], w)

            pltpu.sync_copy(wout_v, w_h.at[pl.ds((base + c0) * 16,
                                                 CHU * 16)])

        @pl.loop(0, NCH, step=2)
        def _pair(g):
            for b in range(2):
                gg = g + b
                c0 = gg * CHU

                @pl.when(g >= 2)
                def _drain():
                    pltpu.make_async_copy(
                        rows[b], gx_h.at[pl.ds(0, CHU), :],
                        sout[b]).wait()

                pltpu.sync_copy(idx_h.at[pl.ds(base + c0, CHU)], idxc[b])
                in_h = pltpu.async_copy(feat_h.at[idxc[b]], rows[b],
                                        sin[b])
                weights_for(c0)
                in_h.wait()
                pltpu.async_copy(rows[b], gx_h.at[pl.ds(base + c0, CHU), :],
                                 sout[b])

        for b in range(2):
            pltpu.make_async_copy(
                rows[b], gx_h.at[pl.ds(0, CHU), :], sout[b]).wait()

    return fused(feats, px, py, pz, idx_flat, kp_rep)


def _tc_body(gx_ref, wt_ref, w2_ref, out_ref):
    feats = gx_ref[...]
    wfs = []
    for k in range(K):
        wcol = wt_ref[:, k:k + 1]
        scaled = feats * wcol
        wfs.append(scaled.reshape(B, H, D_IN).sum(axis=1))
    wf = jnp.concatenate(wfs, axis=1)
    out = jnp.dot(wf, w2_ref[...], preferred_element_type=jnp.float32)
    out_ref[...] = jnp.where(out > 0, out, 0.1 * out)


def _tc_compute(gx, wt, w2):
    return pl.pallas_call(
        _tc_body,
        grid=(GRID,),
        in_specs=[
            pl.BlockSpec((B * H, D_IN), lambda i: (i, 0)),
            pl.BlockSpec((B * H, 16), lambda i: (i, 0)),
            pl.BlockSpec((K * D_IN, D_OUT), lambda i: (0, 0)),
        ],
        out_specs=pl.BlockSpec((B, D_OUT), lambda i: (i, 0)),
        out_shape=jax.ShapeDtypeStruct((NP, D_OUT), jnp.float32),
    )(gx, wt, w2)


def kernel(points, features, neighbor_indices, kernel_points, weights):
    px = points[:, 0]
    py = points[:, 1]
    pz = points[:, 2]
    idx_pad = jnp.pad(neighbor_indices, ((0, NP - N), (0, 0)))
    idx_flat = idx_pad.reshape(EP)
    kp_rep = jnp.tile(kernel_points.reshape(K * 3, 1), (1, 16)).reshape(720)
    w2 = weights.reshape(K * D_IN, D_OUT)
    gx, wt = _sc_fused(features, px, py, pz, idx_flat, kp_rep)
    out = _tc_compute(gx, wt.reshape(EP, 16), w2)
    return out[:N]


# speedup vs baseline: 4.6231x; 1.9645x over previous
import functools

import jax
import jax.numpy as jnp
from jax import lax
from jax.experimental import pallas as pl
from jax.experimental.pallas import tpu as pltpu
from jax.experimental.pallas import tpu_sc as plsc

N = 10000
H = 32
D_IN = 128
D_OUT = 128
K = 15
KP_EXTENT = 0.12

NP = 10240
EP = NP * H
NW = 32
EW = EP // NW
CHU = 256
NCH = EW // CHU
B = 256
GRID = NP // B

_MESH = dict(core_axis_name="c", subcore_axis_name="s")


def _sc_fused(feats, px, py, pz, idx_flat, kp_rep):
    mesh = plsc.VectorSubcoreMesh(**_MESH)

    @functools.partial(
        pl.kernel,
        out_type=(
            jax.ShapeDtypeStruct((EP, D_IN), jnp.float32),
            jax.ShapeDtypeStruct((16, EP), jnp.float32),
        ),
        mesh=mesh,
        compiler_params=pltpu.CompilerParams(needs_layout_passes=False),
        scratch_types=[
            pltpu.VMEM((N,), jnp.float32),
            pltpu.VMEM((N,), jnp.float32),
            pltpu.VMEM((N,), jnp.float32),
            pltpu.VMEM((EW,), jnp.int32),
            pltpu.VMEM((720,), jnp.float32),
            pltpu.VMEM((16, CHU), jnp.float32),
            pltpu.VMEM((CHU, D_IN), jnp.float32),
            pltpu.VMEM((CHU, D_IN), jnp.float32),
            pltpu.VMEM((CHU,), jnp.int32),
            pltpu.VMEM((CHU,), jnp.int32),
            pltpu.SemaphoreType.DMA,
            pltpu.SemaphoreType.DMA,
            pltpu.SemaphoreType.DMA,
            pltpu.SemaphoreType.DMA,
        ],
    )
    def fused(feat_h, px_h, py_h, pz_h, idx_h, kp_h, gx_h, w_h,
              px_v, py_v, pz_v, idx_v, kp_v, wout_v,
              rows0, rows1, idxc0, idxc1,
              sin0, sin1, sout0, sout1):
        wid = lax.axis_index("s") * 2 + lax.axis_index("c")
        base = wid * EW
        pltpu.sync_copy(px_h, px_v)
        pltpu.sync_copy(py_h, py_v)
        pltpu.sync_copy(pz_h, pz_v)
        pltpu.sync_copy(idx_h.at[pl.ds(base, EW)], idx_v)
        pltpu.sync_copy(kp_h, kp_v)

        kvecs = [
            (kp_v[pl.ds(k * 48, 16)],
             kp_v[pl.ds(k * 48 + 16, 16)],
             kp_v[pl.ds(k * 48 + 32, 16)])
            for k in range(K)
        ]
        lane = lax.iota(jnp.int32, 16)
        inv_ext = jnp.float32(1.0 / KP_EXTENT)
        rows = (rows0, rows1)
        idxc = (idxc0, idxc1)
        sin = (sin0, sin1)
        sout = (sout0, sout1)

        def weights_for(c0):
            @pl.loop(0, CHU, step=16)
            def _vec(v0):
                i_nbr = idx_v[pl.ds(c0 + v0, 16)]
                e_g = base + c0 + v0 + lane
                i_ctr = lax.shift_right_logical(e_g, 5)
                xn = plsc.load_gather(px_v, [i_nbr])
                yn = plsc.load_gather(py_v, [i_nbr])
                zn = plsc.load_gather(pz_v, [i_nbr])
                xc = plsc.load_gather(px_v, [i_ctr])
                yc = plsc.load_gather(py_v, [i_ctr])
                zc = plsc.load_gather(pz_v, [i_ctr])
                rx = xn - xc
                ry = yn - yc
                rz = zn - zc
                row = v0 + lane
                for k in range(K):
                    kx, ky, kz = kvecs[k]
                    dx = rx - kx
                    dy = ry - ky
                    dz = rz - kz
                    d2 = jnp.maximum(dx * dx + dy * dy + dz * dz,
                                     jnp.float32(1e-24))
                    bits = plsc.bitcast(d2, jnp.int32)
                    seed = jnp.int32(0x5F3759DF) - lax.shift_right_logical(
                        bits, 1)
                    r = plsc.bitcast(seed, jnp.float32)
                    for _ in range(3):
                        r = r * (jnp.float32(1.5)
                                 - jnp.float32(0.5) * d2 * r * r)
                    dist = d2 * r
                    w = jnp.maximum(jnp.float32(1.0) - dist * inv_ext,
                                    jnp.float32(0.0))
                    kfull = jnp.full((16,), k, jnp.int32)
                    plsc.store_scatter(wout_v, [kfull, row], w)

            pltpu.sync_copy(wout_v, w_h.at[:, pl.ds(base + c0, CHU)])

        @pl.loop(0, NCH, step=2)
        def _pair(g):
            for b in range(2):
                gg = g + b
                c0 = gg * CHU

                @pl.when(g >= 2)
                def _drain():
                    pltpu.make_async_copy(
                        rows[b], gx_h.at[pl.ds(0, CHU), :],
                        sout[b]).wait()

                pltpu.sync_copy(idx_h.at[pl.ds(base + c0, CHU)], idxc[b])
                in_h = pltpu.async_copy(feat_h.at[idxc[b]], rows[b],
                                        sin[b])
                weights_for(c0)
                in_h.wait()
                pltpu.async_copy(rows[b], gx_h.at[pl.ds(base + c0, CHU), :],
                                 sout[b])

        for b in range(2):
            pltpu.make_async_copy(
                rows[b], gx_h.at[pl.ds(0, CHU), :], sout[b]).wait()

    return fused(feats, px, py, pz, idx_flat, kp_rep)


GP = 32
GE = GP * H
NG = B // GP


def _tc_body(gx_ref, wt_ref, mask_ref, w2_ref, out_ref):
    feats = gx_ref[...].astype(jnp.bfloat16)
    mask = mask_ref[...]
    wfs = []
    for g in range(NG):
        wt_g = wt_ref[:, g * GE:(g + 1) * GE].astype(jnp.bfloat16)
        lhs = jnp.tile(wt_g, (GP, 1)) * mask
        x_g = feats[g * GE:(g + 1) * GE, :]
        acc = jnp.dot(lhs, x_g, preferred_element_type=jnp.float32)
        wfs.append(acc.reshape(GP, 16 * D_IN))
    wf = jnp.concatenate(wfs, axis=0).astype(jnp.bfloat16)
    out = jnp.dot(wf, w2_ref[...], preferred_element_type=jnp.float32)
    out_ref[...] = jnp.where(out > 0, out, 0.1 * out)


def _tc_compute(gx, wt, mask, w2pad):
    return pl.pallas_call(
        _tc_body,
        grid=(GRID,),
        in_specs=[
            pl.BlockSpec((B * H, D_IN), lambda i: (i, 0)),
            pl.BlockSpec((16, B * H), lambda i: (0, i)),
            pl.BlockSpec((GP * 16, GE), lambda i: (0, 0)),
            pl.BlockSpec((16 * D_IN, D_OUT), lambda i: (0, 0)),
        ],
        out_specs=pl.BlockSpec((B, D_OUT), lambda i: (i, 0)),
        out_shape=jax.ShapeDtypeStruct((NP, D_OUT), jnp.float32),
    )(gx, wt, mask, w2pad)


def kernel(points, features, neighbor_indices, kernel_points, weights):
    px = points[:, 0]
    py = points[:, 1]
    pz = points[:, 2]
    idx_pad = jnp.pad(neighbor_indices, ((0, NP - N), (0, 0)))
    idx_flat = idx_pad.reshape(EP)
    kp_rep = jnp.tile(kernel_points.reshape(K * 3, 1), (1, 16)).reshape(720)
    w2pad = jnp.concatenate(
        [weights.reshape(K * D_IN, D_OUT),
         jnp.zeros((D_IN, D_OUT), jnp.float32)]).astype(jnp.bfloat16)
    mask = (jnp.arange(GP * 16)[:, None] // 16
            == jnp.arange(GE)[None, :] // H).astype(jnp.bfloat16)
    gx, wt = _sc_fused(features, px, py, pz, idx_flat, kp_rep)
    out = _tc_compute(gx, wt, mask, w2pad)
    return out[:N]
